# baseline (speedup 1.0000x reference)
import functools

import jax
import jax.numpy as jnp
from jax import lax
from jax.experimental import pallas as pl
from jax.experimental.pallas import tpu as pltpu
from jax.experimental.pallas import tpu_sc as plsc

B = 16384
E = 32
PACK = 128 // E
NC, NS = 2, 16
NW = NC * NS
BPW = B // NW
CHUNK = 128
NCH = BPW // CHUNK

BLK = 2048


def _sc_gather_wide(ut_r, it_r, uidx, iidx):
    mesh = plsc.VectorSubcoreMesh(core_axis_name="c", subcore_axis_name="s")
    out_type = (jax.ShapeDtypeStruct((B, 128), jnp.float32),
                jax.ShapeDtypeStruct((B, 128), jnp.float32))

    @functools.partial(
        pl.kernel, mesh=mesh, out_type=out_type,
        scratch_types=[
            pltpu.VMEM((CHUNK,), jnp.int32),
            pltpu.VMEM((CHUNK,), jnp.int32),
            pltpu.VMEM((CHUNK, 128), jnp.float32),
            pltpu.VMEM((CHUNK, 128), jnp.float32),
            pltpu.SemaphoreType.DMA,
            pltpu.SemaphoreType.DMA,
        ],
    )
    def gather_kernel(ut_hbm, it_hbm, ui_hbm, ii_hbm, uo_hbm, io_hbm,
                      uidx_v, iidx_v, ubuf, ibuf, usem, isem):
        wid = lax.axis_index("s") * NC + lax.axis_index("c")

        @pl.loop(0, NCH)
        def _(c):
            base = pl.multiple_of(wid * BPW + c * CHUNK, CHUNK)
            pltpu.sync_copy(ui_hbm.at[pl.ds(base, CHUNK)], uidx_v)
            pltpu.sync_copy(ii_hbm.at[pl.ds(base, CHUNK)], iidx_v)
            cu = pltpu.async_copy(ut_hbm.at[uidx_v], ubuf, usem)
            ci = pltpu.async_copy(it_hbm.at[iidx_v], ibuf, isem)
            cu.wait()
            ci.wait()
            pltpu.sync_copy(ubuf, uo_hbm.at[pl.ds(base, CHUNK)])
            pltpu.sync_copy(ibuf, io_hbm.at[pl.ds(base, CHUNK)])

    return gather_kernel(ut_r, it_r, uidx, iidx)


def _mlp_body(uw_ref, iw_ref, uo_ref, io_ref,
              w1m_ref, b1m_ref, w2m_ref, b2m_ref, w3m_ref, b3m_ref,
              w1a_ref, b1a_ref, w2a_ref, b2a_ref, w3a_ref, b3a_ref,
              ctr_ref, ctcvr_ref):
    prec = lax.Precision.HIGHEST
    uw = uw_ref[...]
    iw = iw_ref[...]
    uo = uo_ref[...]
    io = io_ref[...]
    u = jnp.zeros((uw.shape[0], E), jnp.float32)
    it = jnp.zeros((iw.shape[0], E), jnp.float32)
    for k in range(PACK):
        u = jnp.where(uo == k, uw[:, k * E:(k + 1) * E], u)
        it = jnp.where(io == k, iw[:, k * E:(k + 1) * E], it)
    v = jnp.concatenate([u, it], axis=1)
    h = jnp.maximum(jnp.dot(v, w1m_ref[...], precision=prec) + b1m_ref[...], 0.0)
    h = jnp.maximum(jnp.dot(h, w2m_ref[...], precision=prec) + b2m_ref[...], 0.0)
    cvr = jax.nn.sigmoid(
        jnp.sum(h * w3m_ref[...], axis=1, keepdims=True) + b3m_ref[...])
    g = jnp.maximum(jnp.dot(v, w1a_ref[...], precision=prec) + b1a_ref[...], 0.0)
    g = jnp.maximum(jnp.dot(g, w2a_ref[...], precision=prec) + b2a_ref[...], 0.0)
    ctr = jax.nn.sigmoid(
        jnp.sum(g * w3a_ref[...], axis=1, keepdims=True) + b3a_ref[...])
    ctr_ref[...] = ctr
    ctcvr_ref[...] = ctr * cvr


def _mlp(u_wide, it_wide, u_off, it_off, W1m, b1m, W2m, b2m, W3m, b3m,
         W1a, b1a, W2a, b2a, W3a, b3a):
    M = W1m.shape[1]
    A = W1a.shape[1]
    full = lambda shape: pl.BlockSpec(shape, lambda i: (0, 0))
    grid_spec = pl.GridSpec(
        grid=(B // BLK,),
        in_specs=[
            pl.BlockSpec((BLK, 128), lambda i: (i, 0)),
            pl.BlockSpec((BLK, 128), lambda i: (i, 0)),
            pl.BlockSpec((BLK, 1), lambda i: (i, 0)),
            pl.BlockSpec((BLK, 1), lambda i: (i, 0)),
            full((2 * E, M)), full((1, M)), full((M, M)), full((1, M)),
            full((1, M)), full((1, 1)),
            full((2 * E, A)), full((1, A)), full((A, A)), full((1, A)),
            full((1, A)), full((1, 1)),
        ],
        out_specs=(
            pl.BlockSpec((BLK, 1), lambda i: (i, 0)),
            pl.BlockSpec((BLK, 1), lambda i: (i, 0)),
        ),
    )
    return pl.pallas_call(
        _mlp_body,
        grid_spec=grid_spec,
        out_shape=(jax.ShapeDtypeStruct((B, 1), jnp.float32),
                   jax.ShapeDtypeStruct((B, 1), jnp.float32)),
    )(u_wide, it_wide, u_off, it_off,
      W1m, b1m.reshape(1, M), W2m, b2m.reshape(1, M),
      W3m.reshape(1, M), b3m.reshape(1, 1),
      W1a, b1a.reshape(1, A), W2a, b2a.reshape(1, A),
      W3a.reshape(1, A), b3a.reshape(1, 1))


def kernel(user, item, user_table, item_table, W1m, b1m, W2m, b2m, W3m, b3m,
           W1a, b1a, W2a, b2a, W3a, b3a):
    user = user.astype(jnp.int32)
    item = item.astype(jnp.int32)
    U = user_table.shape[0]
    I = item_table.shape[0]
    ut_r = user_table.reshape(U // PACK, 128)
    it_r = item_table.reshape(I // PACK, 128)
    u_wide, it_wide = _sc_gather_wide(ut_r, it_r, user // PACK, item // PACK)
    ctr, ctcvr = _mlp(u_wide, it_wide,
                      (user % PACK).reshape(B, 1), (item % PACK).reshape(B, 1),
                      W1m, b1m, W2m, b2m, W3m, b3m,
                      W1a, b1a, W2a, b2a, W3a, b3a)
    return ctr, ctcvr
